# Initial kernel scaffold; baseline (speedup 1.0000x reference)
import jax
import jax.numpy as jnp
from jax.experimental import pallas as pl


def kernel(x, edge_index, W, b):
    raise NotImplementedError("write your pallas kernel here")


# speedup vs baseline: 12.7554x; 12.7554x over previous
import functools

import jax
import jax.numpy as jnp
from jax import lax
from jax.experimental import pallas as pl
from jax.experimental.pallas import tpu as pltpu
from jax.experimental.pallas import tpu_sc as plsc

N = 10000
E = 320000
DIN = 128
DOUT = 64
HALF = DOUT // 2

N_PAD = 10240
NC = 2
NT = 16
CHUNK = 128
KPT = -(-E // (NT * CHUNK))
E_PAD = NT * CHUNK * KPT
NPT = N_PAD // NT

_mesh = plsc.VectorSubcoreMesh(core_axis_name="c", subcore_axis_name="s")


@functools.partial(
    pl.kernel,
    out_type=jax.ShapeDtypeStruct((NC, N_PAD), jnp.float32),
    mesh=_mesh,
    scratch_types=[
        pltpu.VMEM((CHUNK,), jnp.int32),
        pltpu.VMEM((CHUNK,), jnp.float32),
        pltpu.VMEM((NPT,), jnp.float32),
        pltpu.VMEM_SHARED((N_PAD,), jnp.float32),
    ],
)
def _sc_hist(col_hbm, out_hbm, idx_v, ones_v, zer_v, acc_sh):
    cid = lax.axis_index("c")
    wid = lax.axis_index("s")

    def fill(ref, n16, val):
        def body(i, _):
            ref[pl.ds(i * 16, 16)] = jnp.full((16,), val, jnp.float32)
            return 0
        lax.fori_loop(0, n16, body, 0)

    fill(ones_v, CHUNK // 16, 1.0)
    fill(zer_v, NPT // 16, 0.0)

    pltpu.sync_copy(zer_v, acc_sh.at[pl.ds(wid * NPT, NPT)])
    plsc.subcore_barrier()

    k0 = (KPT + 1) // 2
    lo = cid * k0
    hi = jnp.where(cid == 0, k0, KPT)

    def body(j, _):
        pltpu.sync_copy(col_hbm.at[wid, j], idx_v)
        pltpu.sync_copy(ones_v, acc_sh.at[idx_v], add=True)
        return 0

    lax.fori_loop(lo, hi, body, 0)
    plsc.subcore_barrier()
    pltpu.sync_copy(acc_sh.at[pl.ds(wid * NPT, NPT)],
                    out_hbm.at[cid, pl.ds(wid * NPT, NPT)])


@functools.partial(
    pl.kernel,
    out_type=jax.ShapeDtypeStruct((NC, N_PAD, HALF), jnp.float32),
    mesh=_mesh,
    scratch_types=[
        pltpu.VMEM((CHUNK,), jnp.int32),
        pltpu.VMEM((CHUNK,), jnp.int32),
        pltpu.VMEM((CHUNK, HALF), jnp.float32),
        pltpu.VMEM_SHARED((N_PAD, HALF), jnp.float32),
        pltpu.SemaphoreType.DMA,
    ],
    compiler_params=pltpu.CompilerParams(use_tc_tiling_on_sc=False),
)
def _sc_hop(u_hbm, rowg_hbm, cols_hbm, zeros_hbm, out_hbm,
            ridx_v, cidx_v, rows_v, acc_sh, sem):
    cid = lax.axis_index("c")
    wid = lax.axis_index("s")

    pltpu.sync_copy(zeros_hbm.at[pl.ds(wid * NPT, NPT)],
                    acc_sh.at[pl.ds(wid * NPT, NPT)])
    plsc.subcore_barrier()

    def body(j, _):
        pltpu.sync_copy(rowg_hbm.at[cid, wid, j], ridx_v)
        pltpu.sync_copy(cols_hbm.at[wid, j], cidx_v)
        pltpu.async_copy(u_hbm.at[ridx_v], rows_v, sem).wait()
        pltpu.sync_copy(rows_v, acc_sh.at[cidx_v], add=True)
        return 0

    lax.fori_loop(0, KPT, body, 0)
    plsc.subcore_barrier()
    pltpu.sync_copy(acc_sh.at[pl.ds(wid * NPT, NPT)],
                    out_hbm.at[cid, pl.ds(wid * NPT, NPT)])


RB = 2048


def _tc_scale_in(x_ref, w_ref, dg_ref, u0_ref, dinv_ref, dinv2_ref):
    d = dg_ref[0] + dg_ref[1] + 1.0
    di = lax.rsqrt(d)
    y = lax.dot_general(x_ref[...], w_ref[...],
                        (((1,), (1,)), ((), ())),
                        preferred_element_type=jnp.float32)
    u0 = y * di
    u0_ref[0] = u0[:, :HALF]
    u0_ref[1] = u0[:, HALF:]
    dinv_ref[...] = di
    dinv2_ref[...] = di * di


def _tc_scale_mid(s_ref, u_ref, d2_ref, o_ref):
    d2 = d2_ref[...]
    o_ref[0] = (s_ref[0] + u_ref[0]) * d2
    o_ref[1] = (s_ref[1] + u_ref[1]) * d2


def _tc_finish(s_ref, u_ref, di_ref, b_ref, o_ref):
    di = di_ref[...]
    h = jnp.concatenate(
        [(s_ref[0] + u_ref[0]) * di, (s_ref[1] + u_ref[1]) * di], axis=1)
    h = h + b_ref[...]
    m = jnp.max(h, axis=1, keepdims=True)
    lse = jnp.log(jnp.sum(jnp.exp(h - m), axis=1, keepdims=True)) + m
    o_ref[...] = h - lse


_GRID = N_PAD // RB

_spec_u = pl.BlockSpec((NC, RB, HALF), lambda i: (0, i, 0))
_spec_n1 = pl.BlockSpec((RB, 1), lambda i: (i, 0))

_scale_in = pl.pallas_call(
    _tc_scale_in,
    grid=(_GRID,),
    in_specs=[
        pl.BlockSpec((RB, DIN), lambda i: (i, 0)),
        pl.BlockSpec((DOUT, DIN), lambda i: (0, 0)),
        pl.BlockSpec((NC, RB, 1), lambda i: (0, i, 0)),
    ],
    out_specs=[_spec_u, _spec_n1, _spec_n1],
    out_shape=[
        jax.ShapeDtypeStruct((NC, N_PAD, HALF), jnp.float32),
        jax.ShapeDtypeStruct((N_PAD, 1), jnp.float32),
        jax.ShapeDtypeStruct((N_PAD, 1), jnp.float32),
    ],
)

_scale_mid = pl.pallas_call(
    _tc_scale_mid,
    grid=(_GRID,),
    in_specs=[_spec_u, _spec_u, _spec_n1],
    out_specs=_spec_u,
    out_shape=jax.ShapeDtypeStruct((NC, N_PAD, HALF), jnp.float32),
)

_finish = pl.pallas_call(
    _tc_finish,
    grid=(_GRID,),
    in_specs=[_spec_u, _spec_u, _spec_n1,
              pl.BlockSpec((1, DOUT), lambda i: (0, 0))],
    out_specs=pl.BlockSpec((RB, DOUT), lambda i: (i, 0)),
    out_shape=jax.ShapeDtypeStruct((N_PAD, DOUT), jnp.float32),
)


def kernel(x, edge_index, W, b):
    row = edge_index[0]
    col = edge_index[1]

    pad = E_PAD - E
    row_p = jnp.concatenate([row, jnp.full((pad,), N, jnp.int32)])
    col_p = jnp.concatenate([col, jnp.full((pad,), N, jnp.int32)])
    cols3 = col_p.reshape(NT, KPT, CHUNK)
    rowg = jnp.stack([row_p, row_p + N_PAD]).reshape(NC, NT, KPT, CHUNK)

    x_pad = jnp.concatenate(
        [x, jnp.zeros((N_PAD - N, DIN), jnp.float32)], axis=0)
    zeros_mat = jnp.zeros((N_PAD, HALF), jnp.float32)

    degp = _sc_hist(cols3).reshape(NC, N_PAD, 1)
    u0, dinv, dinv2 = _scale_in(x_pad, W, degp)
    s1 = _sc_hop(u0.reshape(NC * N_PAD, HALF), rowg, cols3, zeros_mat)
    u1 = _scale_mid(s1, u0, dinv2)
    s2 = _sc_hop(u1.reshape(NC * N_PAD, HALF), rowg, cols3, zeros_mat)
    out = _finish(s2, u1, dinv, b.reshape(1, DOUT))
    return out[:N]


# speedup vs baseline: 21.7645x; 1.7063x over previous
import functools

import jax
import jax.numpy as jnp
from jax import lax
from jax.experimental import pallas as pl
from jax.experimental.pallas import tpu as pltpu
from jax.experimental.pallas import tpu_sc as plsc

N = 10000
E = 320000
DIN = 128
DOUT = 64
HALF = DOUT // 2

N_PAD = 10240
NC = 2
NT = 16
CHUNK = 128
NSLOT = 8
LOOK = 4
KPT = 160
E_PAD = NT * CHUNK * KPT
NPT = N_PAD // NT

_mesh = plsc.VectorSubcoreMesh(core_axis_name="c", subcore_axis_name="s")


@functools.partial(
    pl.kernel,
    out_type=jax.ShapeDtypeStruct((NC, N_PAD), jnp.float32),
    mesh=_mesh,
    scratch_types=[
        pltpu.VMEM((CHUNK,), jnp.int32),
        pltpu.VMEM((CHUNK,), jnp.float32),
        pltpu.VMEM((NPT,), jnp.float32),
        pltpu.VMEM_SHARED((N_PAD,), jnp.float32),
    ],
)
def _sc_hist(col_hbm, out_hbm, idx_v, ones_v, zer_v, acc_sh):
    cid = lax.axis_index("c")
    wid = lax.axis_index("s")

    def fill(ref, n16, val):
        def body(i, _):
            ref[pl.ds(i * 16, 16)] = jnp.full((16,), val, jnp.float32)
            return 0
        lax.fori_loop(0, n16, body, 0)

    fill(ones_v, CHUNK // 16, 1.0)
    fill(zer_v, NPT // 16, 0.0)

    pltpu.sync_copy(zer_v, acc_sh.at[pl.ds(wid * NPT, NPT)])
    plsc.subcore_barrier()

    k0 = (KPT + 1) // 2
    lo = cid * k0
    hi = jnp.where(cid == 0, k0, KPT)

    def body(j, _):
        pltpu.sync_copy(col_hbm.at[wid, j], idx_v)
        pltpu.sync_copy(ones_v, acc_sh.at[idx_v], add=True)
        return 0

    lax.fori_loop(lo, hi, body, 0)
    plsc.subcore_barrier()
    pltpu.sync_copy(acc_sh.at[pl.ds(wid * NPT, NPT)],
                    out_hbm.at[cid, pl.ds(wid * NPT, NPT)])


@functools.partial(
    pl.kernel,
    out_type=jax.ShapeDtypeStruct((NC, N_PAD, HALF), jnp.float32),
    mesh=_mesh,
    scratch_types=[
        pltpu.VMEM((KPT, CHUNK), jnp.int32),
        pltpu.VMEM((KPT, CHUNK), jnp.int32),
        pltpu.VMEM((NSLOT, CHUNK, HALF), jnp.float32),
        pltpu.VMEM_SHARED((N_PAD, HALF), jnp.float32),
        pltpu.SemaphoreType.DMA((NSLOT,)),
        pltpu.SemaphoreType.DMA((NSLOT,)),
    ],
    compiler_params=pltpu.CompilerParams(use_tc_tiling_on_sc=False),
)
def _sc_hop(u_hbm, rowg_hbm, cols_hbm, zeros_hbm, out_hbm,
            ridx_all, cidx_all, rows_v, acc_sh, gsem, ssem):
    cid = lax.axis_index("c")
    wid = lax.axis_index("s")

    pltpu.sync_copy(rowg_hbm.at[cid, wid], ridx_all)
    pltpu.sync_copy(cols_hbm.at[wid], cidx_all)
    pltpu.sync_copy(zeros_hbm.at[pl.ds(wid * NPT, NPT)],
                    acc_sh.at[pl.ds(wid * NPT, NPT)])
    plsc.subcore_barrier()

    def issue_gather(c, d):
        pltpu.async_copy(u_hbm.at[ridx_all.at[c]], rows_v.at[d], gsem.at[d])

    def wait_gather(d):
        pltpu.make_async_copy(u_hbm.at[pl.ds(0, CHUNK)], rows_v.at[d],
                              gsem.at[d]).wait()

    def issue_scatter(c, d):
        pltpu.async_copy(rows_v.at[d], acc_sh.at[cidx_all.at[c]], ssem.at[d],
                         add=True)

    def wait_scatter(d):
        pltpu.make_async_copy(rows_v.at[d], acc_sh.at[cidx_all.at[0]],
                              ssem.at[d]).wait()

    def step(c, d, wait_s, do_gather):
        d2 = (d + LOOK) % NSLOT
        wait_gather(d)
        issue_scatter(c, d)
        if wait_s:
            wait_scatter(d2)
        if do_gather:
            issue_gather(c + LOOK, d2)

    for i in range(LOOK):
        issue_gather(i, i)

    for d in range(NSLOT):
        step(d, d, wait_s=(d >= LOOK), do_gather=True)

    def group(j, _):
        for d in range(NSLOT):
            step(j * NSLOT + d, d, wait_s=True, do_gather=True)
        return 0

    lax.fori_loop(1, KPT // NSLOT - 1, group, 0)

    for d in range(NSLOT):
        c = KPT - NSLOT + d
        step(c, d, wait_s=(d < LOOK), do_gather=(c + LOOK < KPT))

    for d in range(NSLOT):
        wait_scatter(d)

    plsc.subcore_barrier()
    pltpu.sync_copy(acc_sh.at[pl.ds(wid * NPT, NPT)],
                    out_hbm.at[cid, pl.ds(wid * NPT, NPT)])


RB = 2048


def _tc_scale_in(x_ref, w_ref, dg_ref, u0_ref, dinv_ref, dinv2_ref):
    d = dg_ref[0] + dg_ref[1] + 1.0
    di = lax.rsqrt(d)
    y = lax.dot_general(x_ref[...], w_ref[...],
                        (((1,), (1,)), ((), ())),
                        preferred_element_type=jnp.float32)
    u0 = y * di
    u0_ref[0] = u0[:, :HALF]
    u0_ref[1] = u0[:, HALF:]
    dinv_ref[...] = di
    dinv2_ref[...] = di * di


def _tc_scale_mid(s_ref, u_ref, d2_ref, o_ref):
    d2 = d2_ref[...]
    o_ref[0] = (s_ref[0] + u_ref[0]) * d2
    o_ref[1] = (s_ref[1] + u_ref[1]) * d2


def _tc_finish(s_ref, u_ref, di_ref, b_ref, o_ref):
    di = di_ref[...]
    h = jnp.concatenate(
        [(s_ref[0] + u_ref[0]) * di, (s_ref[1] + u_ref[1]) * di], axis=1)
    h = h + b_ref[...]
    m = jnp.max(h, axis=1, keepdims=True)
    lse = jnp.log(jnp.sum(jnp.exp(h - m), axis=1, keepdims=True)) + m
    o_ref[...] = h - lse


_GRID = N_PAD // RB

_spec_u = pl.BlockSpec((NC, RB, HALF), lambda i: (0, i, 0))
_spec_n1 = pl.BlockSpec((RB, 1), lambda i: (i, 0))

_scale_in = pl.pallas_call(
    _tc_scale_in,
    grid=(_GRID,),
    in_specs=[
        pl.BlockSpec((RB, DIN), lambda i: (i, 0)),
        pl.BlockSpec((DOUT, DIN), lambda i: (0, 0)),
        pl.BlockSpec((NC, RB, 1), lambda i: (0, i, 0)),
    ],
    out_specs=[_spec_u, _spec_n1, _spec_n1],
    out_shape=[
        jax.ShapeDtypeStruct((NC, N_PAD, HALF), jnp.float32),
        jax.ShapeDtypeStruct((N_PAD, 1), jnp.float32),
        jax.ShapeDtypeStruct((N_PAD, 1), jnp.float32),
    ],
)

_scale_mid = pl.pallas_call(
    _tc_scale_mid,
    grid=(_GRID,),
    in_specs=[_spec_u, _spec_u, _spec_n1],
    out_specs=_spec_u,
    out_shape=jax.ShapeDtypeStruct((NC, N_PAD, HALF), jnp.float32),
)

_finish = pl.pallas_call(
    _tc_finish,
    grid=(_GRID,),
    in_specs=[_spec_u, _spec_u, _spec_n1,
              pl.BlockSpec((1, DOUT), lambda i: (0, 0))],
    out_specs=pl.BlockSpec((RB, DOUT), lambda i: (i, 0)),
    out_shape=jax.ShapeDtypeStruct((N_PAD, DOUT), jnp.float32),
)


def kernel(x, edge_index, W, b):
    row = edge_index[0]
    col = edge_index[1]

    pad = E_PAD - E
    row_p = jnp.concatenate([row, jnp.full((pad,), N, jnp.int32)])
    col_p = jnp.concatenate([col, jnp.full((pad,), N, jnp.int32)])
    cols3 = col_p.reshape(NT, KPT, CHUNK)
    rowg = jnp.stack([row_p, row_p + N_PAD]).reshape(NC, NT, KPT, CHUNK)

    x_pad = jnp.concatenate(
        [x, jnp.zeros((N_PAD - N, DIN), jnp.float32)], axis=0)
    zeros_mat = jnp.zeros((N_PAD, HALF), jnp.float32)

    degp = _sc_hist(cols3).reshape(NC, N_PAD, 1)
    u0, dinv, dinv2 = _scale_in(x_pad, W, degp)
    s1 = _sc_hop(u0.reshape(NC * N_PAD, HALF), rowg, cols3, zeros_mat)
    u1 = _scale_mid(s1, u0, dinv2)
    s2 = _sc_hop(u1.reshape(NC * N_PAD, HALF), rowg, cols3, zeros_mat)
    out = _finish(s2, u1, dinv, b.reshape(1, DOUT))
    return out[:N]


# speedup vs baseline: 33.3766x; 1.5335x over previous
import functools

import jax
import jax.numpy as jnp
from jax import lax
from jax.experimental import pallas as pl
from jax.experimental.pallas import tpu as pltpu
from jax.experimental.pallas import tpu_sc as plsc

N = 10000
E = 320000
DIN = 128
DOUT = 64
HALF = DOUT // 2

N_PAD = 10240
NC = 2
NT = 16
CHUNK = 128
NSLOT = 8
LOOK = 4
KPT = 160
E_PAD = NT * CHUNK * KPT
NPT = N_PAD // NT

_mesh = plsc.VectorSubcoreMesh(core_axis_name="c", subcore_axis_name="s")


@functools.partial(
    pl.kernel,
    out_type=jax.ShapeDtypeStruct((NC, N_PAD), jnp.float32),
    mesh=_mesh,
    scratch_types=[
        pltpu.VMEM((CHUNK,), jnp.int32),
        pltpu.VMEM((CHUNK,), jnp.float32),
        pltpu.VMEM((NPT,), jnp.float32),
        pltpu.VMEM_SHARED((N_PAD,), jnp.float32),
    ],
)
def _sc_hist(col_hbm, out_hbm, idx_v, ones_v, zer_v, acc_sh):
    cid = lax.axis_index("c")
    wid = lax.axis_index("s")

    def fill(ref, n16, val):
        def body(i, _):
            ref[pl.ds(i * 16, 16)] = jnp.full((16,), val, jnp.float32)
            return 0
        lax.fori_loop(0, n16, body, 0)

    fill(ones_v, CHUNK // 16, 1.0)
    fill(zer_v, NPT // 16, 0.0)

    pltpu.sync_copy(zer_v, acc_sh.at[pl.ds(wid * NPT, NPT)])
    plsc.subcore_barrier()

    k0 = (KPT + 1) // 2
    lo = cid * k0
    hi = jnp.where(cid == 0, k0, KPT)

    def body(j, _):
        pltpu.sync_copy(col_hbm.at[wid, j], idx_v)
        pltpu.sync_copy(ones_v, acc_sh.at[idx_v], add=True)
        return 0

    lax.fori_loop(lo, hi, body, 0)
    plsc.subcore_barrier()
    pltpu.sync_copy(acc_sh.at[pl.ds(wid * NPT, NPT)],
                    out_hbm.at[cid, pl.ds(wid * NPT, NPT)])


@functools.partial(
    pl.kernel,
    out_type=jax.ShapeDtypeStruct((NC, N_PAD, HALF), jnp.float32),
    mesh=_mesh,
    scratch_types=[
        pltpu.VMEM((KPT, CHUNK), jnp.int32),
        pltpu.VMEM((KPT, CHUNK), jnp.int32),
        pltpu.VMEM((NSLOT, CHUNK, HALF), jnp.float32),
        pltpu.VMEM_SHARED((N_PAD, HALF), jnp.float32),
        pltpu.VMEM_SHARED((N_PAD, HALF), jnp.float32),
        pltpu.SemaphoreType.DMA((NSLOT,)),
        pltpu.SemaphoreType.DMA((NSLOT,)),
    ],
    compiler_params=pltpu.CompilerParams(use_tc_tiling_on_sc=False),
)
def _sc_hop(u_hbm, rows_hbm, cols_hbm, zeros_hbm, out_hbm,
            ridx_all, cidx_all, rows_v, acc_sh, u_sh, gsem, ssem):
    cid = lax.axis_index("c")
    wid = lax.axis_index("s")

    pltpu.sync_copy(rows_hbm.at[wid], ridx_all)
    pltpu.sync_copy(cols_hbm.at[wid], cidx_all)
    pltpu.sync_copy(zeros_hbm.at[pl.ds(wid * NPT, NPT)],
                    acc_sh.at[pl.ds(wid * NPT, NPT)])
    pltpu.sync_copy(u_hbm.at[pl.ds(cid * N_PAD + wid * NPT, NPT)],
                    u_sh.at[pl.ds(wid * NPT, NPT)])
    plsc.subcore_barrier()

    def issue_gather(c, d):
        pltpu.async_copy(u_sh.at[ridx_all.at[c]], rows_v.at[d], gsem.at[d])

    def wait_gather(d):
        pltpu.make_async_copy(u_hbm.at[pl.ds(0, CHUNK)], rows_v.at[d],
                              gsem.at[d]).wait()

    def issue_scatter(c, d):
        pltpu.async_copy(rows_v.at[d], acc_sh.at[cidx_all.at[c]], ssem.at[d],
                         add=True)

    def wait_scatter(d):
        pltpu.make_async_copy(rows_v.at[d], acc_sh.at[cidx_all.at[0]],
                              ssem.at[d]).wait()

    def step(c, d, wait_s, do_gather):
        d2 = (d + LOOK) % NSLOT
        wait_gather(d)
        issue_scatter(c, d)
        if wait_s:
            wait_scatter(d2)
        if do_gather:
            issue_gather(c + LOOK, d2)

    for i in range(LOOK):
        issue_gather(i, i)

    for d in range(NSLOT):
        step(d, d, wait_s=(d >= LOOK), do_gather=True)

    def group(j, _):
        for d in range(NSLOT):
            step(j * NSLOT + d, d, wait_s=True, do_gather=True)
        return 0

    lax.fori_loop(1, KPT // NSLOT - 1, group, 0)

    for d in range(NSLOT):
        c = KPT - NSLOT + d
        step(c, d, wait_s=(d < LOOK), do_gather=(c + LOOK < KPT))

    for d in range(NSLOT):
        wait_scatter(d)

    plsc.subcore_barrier()
    pltpu.sync_copy(acc_sh.at[pl.ds(wid * NPT, NPT)],
                    out_hbm.at[cid, pl.ds(wid * NPT, NPT)])


RB = 2048


def _tc_scale_in(x_ref, w_ref, dg_ref, u0_ref, dinv_ref, dinv2_ref):
    d = dg_ref[0] + dg_ref[1] + 1.0
    di = lax.rsqrt(d)
    y = lax.dot_general(x_ref[...], w_ref[...],
                        (((1,), (1,)), ((), ())),
                        preferred_element_type=jnp.float32)
    u0 = y * di
    u0_ref[0] = u0[:, :HALF]
    u0_ref[1] = u0[:, HALF:]
    dinv_ref[...] = di
    dinv2_ref[...] = di * di


def _tc_scale_mid(s_ref, u_ref, d2_ref, o_ref):
    d2 = d2_ref[...]
    o_ref[0] = (s_ref[0] + u_ref[0]) * d2
    o_ref[1] = (s_ref[1] + u_ref[1]) * d2


def _tc_finish(s_ref, u_ref, di_ref, b_ref, o_ref):
    di = di_ref[...]
    h = jnp.concatenate(
        [(s_ref[0] + u_ref[0]) * di, (s_ref[1] + u_ref[1]) * di], axis=1)
    h = h + b_ref[...]
    m = jnp.max(h, axis=1, keepdims=True)
    lse = jnp.log(jnp.sum(jnp.exp(h - m), axis=1, keepdims=True)) + m
    o_ref[...] = h - lse


_GRID = N_PAD // RB

_spec_u = pl.BlockSpec((NC, RB, HALF), lambda i: (0, i, 0))
_spec_n1 = pl.BlockSpec((RB, 1), lambda i: (i, 0))

_scale_in = pl.pallas_call(
    _tc_scale_in,
    grid=(_GRID,),
    in_specs=[
        pl.BlockSpec((RB, DIN), lambda i: (i, 0)),
        pl.BlockSpec((DOUT, DIN), lambda i: (0, 0)),
        pl.BlockSpec((NC, RB, 1), lambda i: (0, i, 0)),
    ],
    out_specs=[_spec_u, _spec_n1, _spec_n1],
    out_shape=[
        jax.ShapeDtypeStruct((NC, N_PAD, HALF), jnp.float32),
        jax.ShapeDtypeStruct((N_PAD, 1), jnp.float32),
        jax.ShapeDtypeStruct((N_PAD, 1), jnp.float32),
    ],
)

_scale_mid = pl.pallas_call(
    _tc_scale_mid,
    grid=(_GRID,),
    in_specs=[_spec_u, _spec_u, _spec_n1],
    out_specs=_spec_u,
    out_shape=jax.ShapeDtypeStruct((NC, N_PAD, HALF), jnp.float32),
)

_finish = pl.pallas_call(
    _tc_finish,
    grid=(_GRID,),
    in_specs=[_spec_u, _spec_u, _spec_n1,
              pl.BlockSpec((1, DOUT), lambda i: (0, 0))],
    out_specs=pl.BlockSpec((RB, DOUT), lambda i: (i, 0)),
    out_shape=jax.ShapeDtypeStruct((N_PAD, DOUT), jnp.float32),
)


def kernel(x, edge_index, W, b):
    row = edge_index[0]
    col = edge_index[1]

    pad = E_PAD - E
    row_p = jnp.concatenate([row, jnp.full((pad,), N, jnp.int32)])
    col_p = jnp.concatenate([col, jnp.full((pad,), N, jnp.int32)])
    cols3 = col_p.reshape(NT, KPT, CHUNK)
    rows3 = row_p.reshape(NT, KPT, CHUNK)

    x_pad = jnp.concatenate(
        [x, jnp.zeros((N_PAD - N, DIN), jnp.float32)], axis=0)
    zeros_mat = jnp.zeros((N_PAD, HALF), jnp.float32)

    degp = _sc_hist(cols3).reshape(NC, N_PAD, 1)
    u0, dinv, dinv2 = _scale_in(x_pad, W, degp)
    s1 = _sc_hop(u0.reshape(NC * N_PAD, HALF), rows3, cols3, zeros_mat)
    u1 = _scale_mid(s1, u0, dinv2)
    s2 = _sc_hop(u1.reshape(NC * N_PAD, HALF), rows3, cols3, zeros_mat)
    out = _finish(s2, u1, dinv, b.reshape(1, DOUT))
    return out[:N]


# speedup vs baseline: 37.5999x; 1.1265x over previous
import functools

import jax
import jax.numpy as jnp
from jax import lax
from jax.experimental import pallas as pl
from jax.experimental.pallas import tpu as pltpu
from jax.experimental.pallas import tpu_sc as plsc

N = 10000
E = 320000
DIN = 128
DOUT = 64
HALF = DOUT // 2

N_PAD = 10240
NC = 2
NT = 16
CHUNK = 128
NSLOT = 8
LOOK = 4
KPT = 160
E_PAD = NT * CHUNK * KPT
NPT = N_PAD // NT

_mesh = plsc.VectorSubcoreMesh(core_axis_name="c", subcore_axis_name="s")


@functools.partial(
    pl.kernel,
    out_type=jax.ShapeDtypeStruct((NC, N_PAD), jnp.float32),
    mesh=_mesh,
    scratch_types=[
        pltpu.VMEM((KPT // 2, CHUNK), jnp.int32),
        pltpu.VMEM((CHUNK,), jnp.float32),
        pltpu.VMEM((NPT,), jnp.float32),
        pltpu.VMEM_SHARED((N_PAD,), jnp.float32),
        pltpu.SemaphoreType.DMA((NSLOT,)),
    ],
)
def _sc_hist(col_hbm, out_hbm, cidx_all, ones_v, zer_v, acc_sh, ssem):
    cid = lax.axis_index("c")
    wid = lax.axis_index("s")
    khalf = KPT // 2

    def fill(ref, n16, val):
        def body(i, _):
            ref[pl.ds(i * 16, 16)] = jnp.full((16,), val, jnp.float32)
            return 0
        lax.fori_loop(0, n16, body, 0)

    fill(ones_v, CHUNK // 16, 1.0)
    fill(zer_v, NPT // 16, 0.0)

    pltpu.sync_copy(col_hbm.at[cid, wid], cidx_all)
    pltpu.sync_copy(zer_v, acc_sh.at[pl.ds(wid * NPT, NPT)])
    plsc.subcore_barrier()

    def issue(c, d):
        pltpu.async_copy(ones_v, acc_sh.at[cidx_all.at[c]], ssem.at[d],
                         add=True)

    def wait(d):
        pltpu.make_async_copy(ones_v, acc_sh.at[cidx_all.at[0]],
                              ssem.at[d]).wait()

    for d in range(NSLOT):
        issue(d, d)

    def group(j, _):
        for d in range(NSLOT):
            wait(d)
            issue(j * NSLOT + d, d)
        return 0

    lax.fori_loop(1, khalf // NSLOT, group, 0)
    for d in range(NSLOT):
        wait(d)

    plsc.subcore_barrier()
    pltpu.sync_copy(acc_sh.at[pl.ds(wid * NPT, NPT)],
                    out_hbm.at[cid, pl.ds(wid * NPT, NPT)])


@functools.partial(
    pl.kernel,
    out_type=jax.ShapeDtypeStruct((NC, N_PAD, HALF), jnp.float32),
    mesh=_mesh,
    scratch_types=[
        pltpu.VMEM((KPT, CHUNK), jnp.int32),
        pltpu.VMEM((KPT, CHUNK), jnp.int32),
        pltpu.VMEM((NSLOT, CHUNK, HALF), jnp.float32),
        pltpu.VMEM_SHARED((N_PAD, HALF), jnp.float32),
        pltpu.VMEM_SHARED((N_PAD, HALF), jnp.float32),
        pltpu.SemaphoreType.DMA((NSLOT,)),
        pltpu.SemaphoreType.DMA((NSLOT,)),
    ],
    compiler_params=pltpu.CompilerParams(use_tc_tiling_on_sc=False),
)
def _sc_hop(u_hbm, rows_hbm, cols_hbm, zeros_hbm, out_hbm,
            ridx_all, cidx_all, rows_v, acc_sh, u_sh, gsem, ssem):
    cid = lax.axis_index("c")
    wid = lax.axis_index("s")

    pltpu.sync_copy(rows_hbm.at[wid], ridx_all)
    pltpu.sync_copy(cols_hbm.at[wid], cidx_all)
    pltpu.sync_copy(zeros_hbm.at[pl.ds(wid * NPT, NPT)],
                    acc_sh.at[pl.ds(wid * NPT, NPT)])
    pltpu.sync_copy(u_hbm.at[pl.ds(cid * N_PAD + wid * NPT, NPT)],
                    u_sh.at[pl.ds(wid * NPT, NPT)])
    plsc.subcore_barrier()

    def issue_gather(c, d):
        pltpu.async_copy(u_sh.at[ridx_all.at[c]], rows_v.at[d], gsem.at[d])

    def wait_gather(d):
        pltpu.make_async_copy(u_hbm.at[pl.ds(0, CHUNK)], rows_v.at[d],
                              gsem.at[d]).wait()

    def issue_scatter(c, d):
        pltpu.async_copy(rows_v.at[d], acc_sh.at[cidx_all.at[c]], ssem.at[d],
                         add=True)

    def wait_scatter(d):
        pltpu.make_async_copy(rows_v.at[d], acc_sh.at[cidx_all.at[0]],
                              ssem.at[d]).wait()

    def step(c, d, wait_s, do_gather):
        d2 = (d + LOOK) % NSLOT
        wait_gather(d)
        issue_scatter(c, d)
        if wait_s:
            wait_scatter(d2)
        if do_gather:
            issue_gather(c + LOOK, d2)

    for i in range(LOOK):
        issue_gather(i, i)

    for d in range(NSLOT):
        step(d, d, wait_s=(d >= LOOK), do_gather=True)

    def group(j, _):
        for d in range(NSLOT):
            step(j * NSLOT + d, d, wait_s=True, do_gather=True)
        return 0

    lax.fori_loop(1, KPT // NSLOT - 1, group, 0)

    for d in range(NSLOT):
        c = KPT - NSLOT + d
        step(c, d, wait_s=(d < LOOK), do_gather=(c + LOOK < KPT))

    for d in range(NSLOT):
        wait_scatter(d)

    plsc.subcore_barrier()
    pltpu.sync_copy(acc_sh.at[pl.ds(wid * NPT, NPT)],
                    out_hbm.at[cid, pl.ds(wid * NPT, NPT)])


RB = 2048


def _tc_scale_in(x_ref, w_ref, dg_ref, u0_ref, dinv_ref, dinv2_ref):
    d = dg_ref[0] + dg_ref[1] + 1.0
    di = lax.rsqrt(d)
    y = lax.dot_general(x_ref[...], w_ref[...],
                        (((1,), (1,)), ((), ())),
                        preferred_element_type=jnp.float32)
    u0 = y * di
    u0_ref[0] = u0[:, :HALF]
    u0_ref[1] = u0[:, HALF:]
    dinv_ref[...] = di
    dinv2_ref[...] = di * di


def _tc_scale_mid(s_ref, u_ref, d2_ref, o_ref):
    d2 = d2_ref[...]
    o_ref[0] = (s_ref[0] + u_ref[0]) * d2
    o_ref[1] = (s_ref[1] + u_ref[1]) * d2


def _tc_finish(s_ref, u_ref, di_ref, b_ref, o_ref):
    di = di_ref[...]
    h = jnp.concatenate(
        [(s_ref[0] + u_ref[0]) * di, (s_ref[1] + u_ref[1]) * di], axis=1)
    h = h + b_ref[...]
    m = jnp.max(h, axis=1, keepdims=True)
    lse = jnp.log(jnp.sum(jnp.exp(h - m), axis=1, keepdims=True)) + m
    o_ref[...] = h - lse


_GRID = N_PAD // RB

_spec_u = pl.BlockSpec((NC, RB, HALF), lambda i: (0, i, 0))
_spec_n1 = pl.BlockSpec((RB, 1), lambda i: (i, 0))

_scale_in = pl.pallas_call(
    _tc_scale_in,
    grid=(_GRID,),
    in_specs=[
        pl.BlockSpec((RB, DIN), lambda i: (i, 0)),
        pl.BlockSpec((DOUT, DIN), lambda i: (0, 0)),
        pl.BlockSpec((NC, RB, 1), lambda i: (0, i, 0)),
    ],
    out_specs=[_spec_u, _spec_n1, _spec_n1],
    out_shape=[
        jax.ShapeDtypeStruct((NC, N_PAD, HALF), jnp.float32),
        jax.ShapeDtypeStruct((N_PAD, 1), jnp.float32),
        jax.ShapeDtypeStruct((N_PAD, 1), jnp.float32),
    ],
)

_scale_mid = pl.pallas_call(
    _tc_scale_mid,
    grid=(_GRID,),
    in_specs=[_spec_u, _spec_u, _spec_n1],
    out_specs=_spec_u,
    out_shape=jax.ShapeDtypeStruct((NC, N_PAD, HALF), jnp.float32),
)

_finish = pl.pallas_call(
    _tc_finish,
    grid=(_GRID,),
    in_specs=[_spec_u, _spec_u, _spec_n1,
              pl.BlockSpec((1, DOUT), lambda i: (0, 0))],
    out_specs=pl.BlockSpec((RB, DOUT), lambda i: (i, 0)),
    out_shape=jax.ShapeDtypeStruct((N_PAD, DOUT), jnp.float32),
)


def kernel(x, edge_index, W, b):
    row = edge_index[0]
    col = edge_index[1]

    pad = E_PAD - E
    row_p = jnp.concatenate([row, jnp.full((pad,), N, jnp.int32)])
    col_p = jnp.concatenate([col, jnp.full((pad,), N, jnp.int32)])
    cols3 = col_p.reshape(NT, KPT, CHUNK)
    rows3 = row_p.reshape(NT, KPT, CHUNK)

    x_pad = jnp.concatenate(
        [x, jnp.zeros((N_PAD - N, DIN), jnp.float32)], axis=0)
    zeros_mat = jnp.zeros((N_PAD, HALF), jnp.float32)

    degp = _sc_hist(col_p.reshape(NC, NT, KPT // 2, CHUNK)).reshape(
        NC, N_PAD, 1)
    u0, dinv, dinv2 = _scale_in(x_pad, W, degp)
    s1 = _sc_hop(u0.reshape(NC * N_PAD, HALF), rows3, cols3, zeros_mat)
    u1 = _scale_mid(s1, u0, dinv2)
    s2 = _sc_hop(u1.reshape(NC * N_PAD, HALF), rows3, cols3, zeros_mat)
    out = _finish(s2, u1, dinv, b.reshape(1, DOUT))
    return out[:N]


# speedup vs baseline: 37.8109x; 1.0056x over previous
import functools

import jax
import jax.numpy as jnp
from jax import lax
from jax.experimental import pallas as pl
from jax.experimental.pallas import tpu as pltpu
from jax.experimental.pallas import tpu_sc as plsc

N = 10000
E = 320000
DIN = 128
DOUT = 64
HALF = DOUT // 2

N_PAD = 10240
NC = 2
NT = 16
CHUNK = 128
NSLOT = 8
LOOK = 4
KPT = 160
E_PAD = NT * CHUNK * KPT
NPT = N_PAD // NT

_mesh = plsc.VectorSubcoreMesh(core_axis_name="c", subcore_axis_name="s")


@functools.partial(
    pl.kernel,
    out_type=jax.ShapeDtypeStruct((NC, N_PAD), jnp.float32),
    mesh=_mesh,
    scratch_types=[
        pltpu.VMEM((KPT // 2, CHUNK), jnp.int32),
        pltpu.VMEM((CHUNK,), jnp.float32),
        pltpu.VMEM((NPT,), jnp.float32),
        pltpu.VMEM_SHARED((N_PAD,), jnp.float32),
        pltpu.SemaphoreType.DMA((NSLOT,)),
    ],
)
def _sc_hist(col_hbm, out_hbm, cidx_all, ones_v, zer_v, acc_sh, ssem):
    cid = lax.axis_index("c")
    wid = lax.axis_index("s")
    khalf = KPT // 2

    def fill(ref, n16, val):
        def body(i, _):
            ref[pl.ds(i * 16, 16)] = jnp.full((16,), val, jnp.float32)
            return 0
        lax.fori_loop(0, n16, body, 0)

    fill(ones_v, CHUNK // 16, 1.0)
    fill(zer_v, NPT // 16, 0.0)

    pltpu.sync_copy(col_hbm.at[cid, wid], cidx_all)
    pltpu.sync_copy(zer_v, acc_sh.at[pl.ds(wid * NPT, NPT)])
    plsc.subcore_barrier()

    def issue(c, d):
        pltpu.async_copy(ones_v, acc_sh.at[cidx_all.at[c]], ssem.at[d],
                         add=True)

    def wait(d):
        pltpu.make_async_copy(ones_v, acc_sh.at[cidx_all.at[0]],
                              ssem.at[d]).wait()

    for d in range(NSLOT):
        issue(d, d)

    def group(j, _):
        for d in range(NSLOT):
            wait(d)
            issue(j * NSLOT + d, d)
        return 0

    lax.fori_loop(1, khalf // NSLOT, group, 0)
    for d in range(NSLOT):
        wait(d)

    plsc.subcore_barrier()
    pltpu.sync_copy(acc_sh.at[pl.ds(wid * NPT, NPT)],
                    out_hbm.at[cid, pl.ds(wid * NPT, NPT)])


@functools.partial(
    pl.kernel,
    out_type=jax.ShapeDtypeStruct((NC, N_PAD, HALF), jnp.float32),
    mesh=_mesh,
    scratch_types=[
        pltpu.VMEM((KPT, CHUNK), jnp.int32),
        pltpu.VMEM((KPT, CHUNK), jnp.int32),
        pltpu.VMEM((NSLOT, CHUNK, HALF), jnp.float32),
        pltpu.VMEM_SHARED((N_PAD, HALF), jnp.float32),
        pltpu.VMEM_SHARED((N_PAD, HALF), jnp.float32),
        pltpu.SemaphoreType.DMA((NSLOT,)),
        pltpu.SemaphoreType.DMA((NSLOT,)),
    ],
    compiler_params=pltpu.CompilerParams(use_tc_tiling_on_sc=False),
)
def _sc_hop(u_hbm, rows_hbm, cols_hbm, zeros_hbm, out_hbm,
            ridx_all, cidx_all, rows_v, acc_sh, u_sh, gsem, ssem):
    cid = lax.axis_index("c")
    wid = lax.axis_index("s")

    pltpu.sync_copy(rows_hbm.at[wid], ridx_all)
    pltpu.sync_copy(cols_hbm.at[wid], cidx_all)
    pltpu.sync_copy(zeros_hbm.at[pl.ds(wid * NPT, NPT)],
                    acc_sh.at[pl.ds(wid * NPT, NPT)])
    pltpu.sync_copy(u_hbm.at[pl.ds(cid * N_PAD + wid * NPT, NPT)],
                    u_sh.at[pl.ds(wid * NPT, NPT)])
    plsc.subcore_barrier()

    def issue_gather(c, d):
        pltpu.async_copy(u_sh.at[ridx_all.at[c]], rows_v.at[d], gsem.at[d])

    def wait_gather(d):
        pltpu.make_async_copy(u_hbm.at[pl.ds(0, CHUNK)], rows_v.at[d],
                              gsem.at[d]).wait()

    def issue_scatter(c, d):
        pltpu.async_copy(rows_v.at[d], acc_sh.at[cidx_all.at[c]], ssem.at[d],
                         add=True)

    def wait_scatter(d):
        pltpu.make_async_copy(rows_v.at[d], acc_sh.at[cidx_all.at[0]],
                              ssem.at[d]).wait()

    def step(c, d, wait_s, do_gather):
        d2 = (d + LOOK) % NSLOT
        wait_gather(d)
        issue_scatter(c, d)
        if wait_s:
            wait_scatter(d2)
        if do_gather:
            issue_gather(c + LOOK, d2)

    for i in range(LOOK):
        issue_gather(i, i)

    for d in range(NSLOT):
        step(d, d, wait_s=(d >= LOOK), do_gather=True)

    def group(j, _):
        for d in range(NSLOT):
            step(j * NSLOT + d, d, wait_s=True, do_gather=True)
        return 0

    lax.fori_loop(1, KPT // NSLOT - 1, group, 0)

    for d in range(NSLOT):
        c = KPT - NSLOT + d
        step(c, d, wait_s=(d < LOOK), do_gather=(c + LOOK < KPT))

    for d in range(NSLOT):
        wait_scatter(d)

    plsc.subcore_barrier()
    pltpu.sync_copy(acc_sh.at[pl.ds(wid * NPT, NPT)],
                    out_hbm.at[cid, pl.ds(wid * NPT, NPT)])


RB = 2048


def _tc_scale_in(x_ref, w_ref, dg_ref, u0_ref, dinv_ref, dinv2_ref):
    d = dg_ref[0] + dg_ref[1] + 1.0
    di = lax.rsqrt(d)
    y = lax.dot_general(x_ref[...], w_ref[...],
                        (((1,), (1,)), ((), ())),
                        preferred_element_type=jnp.float32)
    u0 = y * di
    u0_ref[0] = u0[:, :HALF]
    u0_ref[1] = u0[:, HALF:]
    dinv_ref[...] = di
    dinv2_ref[...] = di * di


def _tc_scale_mid(s_ref, u_ref, d2_ref, o_ref):
    d2 = d2_ref[...]
    o_ref[0] = (s_ref[0] + u_ref[0]) * d2
    o_ref[1] = (s_ref[1] + u_ref[1]) * d2


def _tc_finish(s_ref, u_ref, di_ref, b_ref, o_ref):
    di = di_ref[...]
    h = jnp.concatenate(
        [(s_ref[0] + u_ref[0]) * di, (s_ref[1] + u_ref[1]) * di], axis=1)
    h = h + b_ref[...]
    m = jnp.max(h, axis=1, keepdims=True)
    lse = jnp.log(jnp.sum(jnp.exp(h - m), axis=1, keepdims=True)) + m
    o_ref[...] = h - lse


_GRID = N_PAD // RB

_spec_u = pl.BlockSpec((NC, RB, HALF), lambda i: (0, i, 0))
_spec_n1 = pl.BlockSpec((RB, 1), lambda i: (i, 0))

_scale_in = pl.pallas_call(
    _tc_scale_in,
    grid=(_GRID,),
    in_specs=[
        pl.BlockSpec((RB, DIN), lambda i: (i, 0)),
        pl.BlockSpec((DOUT, DIN), lambda i: (0, 0)),
        pl.BlockSpec((NC, RB, 1), lambda i: (0, i, 0)),
    ],
    out_specs=[_spec_u, _spec_n1, _spec_n1],
    out_shape=[
        jax.ShapeDtypeStruct((NC, N_PAD, HALF), jnp.float32),
        jax.ShapeDtypeStruct((N_PAD, 1), jnp.float32),
        jax.ShapeDtypeStruct((N_PAD, 1), jnp.float32),
    ],
)

_scale_mid = pl.pallas_call(
    _tc_scale_mid,
    grid=(_GRID,),
    in_specs=[_spec_u, _spec_u, _spec_n1],
    out_specs=_spec_u,
    out_shape=jax.ShapeDtypeStruct((NC, N_PAD, HALF), jnp.float32),
)

RBF = 2000

_finish = pl.pallas_call(
    _tc_finish,
    grid=(N // RBF,),
    in_specs=[pl.BlockSpec((NC, RBF, HALF), lambda i: (0, i, 0)),
              pl.BlockSpec((NC, RBF, HALF), lambda i: (0, i, 0)),
              pl.BlockSpec((RBF, 1), lambda i: (i, 0)),
              pl.BlockSpec((1, DOUT), lambda i: (0, 0))],
    out_specs=pl.BlockSpec((RBF, DOUT), lambda i: (i, 0)),
    out_shape=jax.ShapeDtypeStruct((N, DOUT), jnp.float32),
)


def kernel(x, edge_index, W, b):
    row = edge_index[0]
    col = edge_index[1]

    pad = E_PAD - E
    row_p = jnp.concatenate([row, jnp.full((pad,), N, jnp.int32)])
    col_p = jnp.concatenate([col, jnp.full((pad,), N, jnp.int32)])
    cols3 = col_p.reshape(NT, KPT, CHUNK)
    rows3 = row_p.reshape(NT, KPT, CHUNK)

    zeros_mat = jnp.zeros((N_PAD, HALF), jnp.float32)

    degp = _sc_hist(col_p.reshape(NC, NT, KPT // 2, CHUNK)).reshape(
        NC, N_PAD, 1)
    u0, dinv, dinv2 = _scale_in(x, W, degp)
    s1 = _sc_hop(u0.reshape(NC * N_PAD, HALF), rows3, cols3, zeros_mat)
    u1 = _scale_mid(s1, u0, dinv2)
    s2 = _sc_hop(u1.reshape(NC * N_PAD, HALF), rows3, cols3, zeros_mat)
    return _finish(s2, u1, dinv, b.reshape(1, DOUT))
